# baseline (speedup 1.0000x reference)
import jax
import jax.numpy as jnp
import numpy as np
import functools
from jax import lax
from jax.experimental import pallas as pl
from jax.experimental.pallas import tpu as pltpu
from jax.experimental.pallas import tpu_sc as plsc

_ATOMIC_NUMBERS = np.array(
    sorted({1, 2, 4, 5, 6, 7, 8, 9, 12, 14, 15, 16, 17, 18, 20, 22, 30, 31,
            32, 33, 34, 35}),
    dtype=np.int32,
)
_NUM_TYPES = 22
_SCALING = 1.5
_N_NODES = 100000

_NW_USED = 10
_NODES_PW = _N_NODES // _NW_USED
_UNROLL = 5
_STEPS_PW = _NODES_PW // (16 * _UNROLL)

_LOOKUP = np.zeros((40,), dtype=np.int32)
_LOOKUP[_ATOMIC_NUMBERS] = np.arange(_NUM_TYPES, dtype=np.int32)


def _make_sc_lookup():
    mesh = plsc.VectorSubcoreMesh(
        core_axis_name="c", subcore_axis_name="s", num_cores=1)

    @functools.partial(
        pl.kernel,
        mesh=mesh,
        out_type=jax.ShapeDtypeStruct((_N_NODES,), jnp.int32),
        scratch_types=[
            pltpu.VMEM((_NODES_PW,), jnp.int32),
            pltpu.VMEM((40,), jnp.int32),
            pltpu.VMEM((_NODES_PW,), jnp.int32),
        ],
        compiler_params=pltpu.CompilerParams(needs_layout_passes=False),
    )
    def sc_lookup(node_hbm, tbl_hbm, cls_hbm, nt_v, tbl_v, cls_v):
        wid = lax.axis_index("s")

        @pl.when(wid < _NW_USED)
        def _():
            base = wid * _NODES_PW
            pltpu.sync_copy(node_hbm.at[pl.ds(base, _NODES_PW)], nt_v)
            pltpu.sync_copy(tbl_hbm, tbl_v)

            def step(j, carry):
                for u in range(_UNROLL):
                    o = j * (16 * _UNROLL) + u * 16
                    z = nt_v[pl.ds(o, 16)] + 1
                    cls_v[pl.ds(o, 16)] = plsc.load_gather(tbl_v, [z])
                return carry

            lax.fori_loop(0, _STEPS_PW, step, 0)
            pltpu.sync_copy(cls_v, cls_hbm.at[pl.ds(base, _NODES_PW)])

    return sc_lookup


_SC_LOOKUP = _make_sc_lookup()


def _tc_body(cls_ref, out_ref):
    cls = cls_ref[...].reshape(1, _N_NODES)
    j = lax.broadcasted_iota(jnp.int32, (_NUM_TYPES, 1), 0)
    out_ref[...] = jnp.where(cls == j, jnp.float32(_SCALING), jnp.float32(0.0))


def kernel(node_type, pos):
    del pos
    nt = node_type.astype(jnp.int32).reshape(-1)
    tbl = jnp.asarray(_LOOKUP)
    cls = _SC_LOOKUP(nt, tbl)
    out_t = pl.pallas_call(
        _tc_body,
        out_shape=jax.ShapeDtypeStruct((_NUM_TYPES, _N_NODES), jnp.float32),
    )(cls)
    return out_t.T


# speedup vs baseline: 1.0458x; 1.0458x over previous
import functools

import jax
import jax.numpy as jnp
import numpy as np
from jax import lax
from jax.experimental import pallas as pl
from jax.experimental.pallas import tpu as pltpu
from jax.experimental.pallas import tpu_sc as plsc

_ATOMIC_NUMBERS = np.array(
    sorted({1, 2, 4, 5, 6, 7, 8, 9, 12, 14, 15, 16, 17, 18, 20, 22, 30, 31,
            32, 33, 34, 35}),
    dtype=np.int32,
)
_NUM_TYPES = 22
_SCALING = 1.5
_N_NODES = 100000

_SC_NODES = 8192
_SC_BLOCK = 4096
_NW = 32
_NODES_PW = _SC_NODES // _NW
_GROUPS_PW = _NODES_PW // 16

_LOOKUP = np.zeros((40,), dtype=np.int32)
_LOOKUP[_ATOMIC_NUMBERS] = np.arange(_NUM_TYPES, dtype=np.int32)


def _make_sc_lookup():
    mesh = plsc.VectorSubcoreMesh(core_axis_name="c", subcore_axis_name="s")

    @functools.partial(
        pl.kernel,
        mesh=mesh,
        out_type=jax.ShapeDtypeStruct((_SC_NODES,), jnp.int32),
        scratch_types=[
            pltpu.VMEM((_NODES_PW,), jnp.int32),
            pltpu.VMEM((40,), jnp.int32),
            pltpu.VMEM((_NODES_PW,), jnp.int32),
        ],
        compiler_params=pltpu.CompilerParams(needs_layout_passes=False),
    )
    def sc_lookup(node_hbm, tbl_hbm, cls_hbm, nt_v, tbl_v, cls_v):
        wid = lax.axis_index("s") * 2 + lax.axis_index("c")
        base = wid * _NODES_PW
        pltpu.sync_copy(node_hbm.at[pl.ds(base, _NODES_PW)], nt_v)
        pltpu.sync_copy(tbl_hbm, tbl_v)
        for j in range(_GROUPS_PW):
            o = j * 16
            z = nt_v[pl.ds(o, 16)] + 1
            cls_v[pl.ds(o, 16)] = plsc.load_gather(tbl_v, [z])
        pltpu.sync_copy(cls_v, cls_hbm.at[pl.ds(base, _NODES_PW)])

    return sc_lookup


_SC_LOOKUP = _make_sc_lookup()


def _tc_tail_body(nt_ref, atoms_ref, out_ref):
    z = nt_ref[...].reshape(1, _N_NODES) + 1
    atoms = atoms_ref[...]
    col = lax.broadcasted_iota(jnp.int32, (1, _N_NODES), 1)
    hit = (z == atoms) & (col >= _SC_NODES)
    out_ref[...] = jnp.where(hit, jnp.float32(_SCALING), jnp.float32(0.0))


def _tc_head_body(cls_ref, prev_ref, out_ref):
    del prev_ref
    cls = cls_ref[...].reshape(1, _SC_BLOCK)
    j = lax.broadcasted_iota(jnp.int32, (_NUM_TYPES, 1), 0)
    out_ref[...] = jnp.where(cls == j, jnp.float32(_SCALING), jnp.float32(0.0))


def kernel(node_type, pos):
    del pos
    nt = node_type.astype(jnp.int32).reshape(-1)
    tbl = jnp.asarray(_LOOKUP)
    atoms = jnp.asarray(_ATOMIC_NUMBERS).reshape(_NUM_TYPES, 1)

    cls_head = _SC_LOOKUP(nt[:_SC_NODES], tbl)

    out_tail = pl.pallas_call(
        _tc_tail_body,
        out_shape=jax.ShapeDtypeStruct((_NUM_TYPES, _N_NODES), jnp.float32),
    )(nt, atoms)

    grid = _SC_NODES // _SC_BLOCK
    out_t = pl.pallas_call(
        _tc_head_body,
        grid=(grid,),
        in_specs=[
            pl.BlockSpec((_SC_BLOCK,), lambda i: (i,)),
            pl.BlockSpec((_NUM_TYPES, _SC_BLOCK), lambda i: (0, i)),
        ],
        out_specs=pl.BlockSpec((_NUM_TYPES, _SC_BLOCK), lambda i: (0, i)),
        out_shape=jax.ShapeDtypeStruct((_NUM_TYPES, _N_NODES), jnp.float32),
        input_output_aliases={1: 0},
    )(cls_head, out_tail)
    return out_t.T


# speedup vs baseline: 1.0558x; 1.0095x over previous
import functools

import jax
import jax.numpy as jnp
import numpy as np
from jax import lax
from jax.experimental import pallas as pl
from jax.experimental.pallas import tpu as pltpu
from jax.experimental.pallas import tpu_sc as plsc

_ATOMIC_NUMBERS = np.array(
    sorted({1, 2, 4, 5, 6, 7, 8, 9, 12, 14, 15, 16, 17, 18, 20, 22, 30, 31,
            32, 33, 34, 35}),
    dtype=np.int32,
)
_NUM_TYPES = 22
_SCALING = 1.5
_N_NODES = 100000

_SC_NODES = 4096
_SC_BLOCK = 4096
_NW = 32
_NODES_PW = _SC_NODES // _NW
_GROUPS_PW = _NODES_PW // 16

_LOOKUP = np.zeros((40,), dtype=np.int32)
_LOOKUP[_ATOMIC_NUMBERS] = np.arange(_NUM_TYPES, dtype=np.int32)


def _make_sc_lookup():
    mesh = plsc.VectorSubcoreMesh(core_axis_name="c", subcore_axis_name="s")

    @functools.partial(
        pl.kernel,
        mesh=mesh,
        out_type=jax.ShapeDtypeStruct((_SC_NODES,), jnp.int32),
        scratch_types=[
            pltpu.VMEM((_NODES_PW,), jnp.int32),
            pltpu.VMEM((40,), jnp.int32),
            pltpu.VMEM((_NODES_PW,), jnp.int32),
        ],
        compiler_params=pltpu.CompilerParams(needs_layout_passes=False),
    )
    def sc_lookup(node_hbm, tbl_hbm, cls_hbm, nt_v, tbl_v, cls_v):
        wid = lax.axis_index("s") * 2 + lax.axis_index("c")
        base = wid * _NODES_PW
        pltpu.sync_copy(node_hbm.at[pl.ds(base, _NODES_PW)], nt_v)
        pltpu.sync_copy(tbl_hbm, tbl_v)
        for j in range(_GROUPS_PW):
            o = j * 16
            z = nt_v[pl.ds(o, 16)] + 1
            cls_v[pl.ds(o, 16)] = plsc.load_gather(tbl_v, [z])
        pltpu.sync_copy(cls_v, cls_hbm.at[pl.ds(base, _NODES_PW)])

    return sc_lookup


_SC_LOOKUP = _make_sc_lookup()


def _tc_tail_body(nt_ref, atoms_ref, out_ref):
    z = nt_ref[...].reshape(1, _N_NODES) + 1
    atoms = atoms_ref[...]
    col = lax.broadcasted_iota(jnp.int32, (1, _N_NODES), 1)
    hit = (z == atoms) & (col >= _SC_NODES)
    out_ref[...] = jnp.where(hit, jnp.float32(_SCALING), jnp.float32(0.0))


def _tc_head_body(cls_ref, prev_ref, out_ref):
    del prev_ref
    cls = cls_ref[...].reshape(1, _SC_BLOCK)
    j = lax.broadcasted_iota(jnp.int32, (_NUM_TYPES, 1), 0)
    out_ref[...] = jnp.where(cls == j, jnp.float32(_SCALING), jnp.float32(0.0))


def kernel(node_type, pos):
    del pos
    nt = node_type.astype(jnp.int32).reshape(-1)
    tbl = jnp.asarray(_LOOKUP)
    atoms = jnp.asarray(_ATOMIC_NUMBERS).reshape(_NUM_TYPES, 1)

    cls_head = _SC_LOOKUP(nt[:_SC_NODES], tbl)

    out_tail = pl.pallas_call(
        _tc_tail_body,
        out_shape=jax.ShapeDtypeStruct((_NUM_TYPES, _N_NODES), jnp.float32),
    )(nt, atoms)

    grid = _SC_NODES // _SC_BLOCK
    out_t = pl.pallas_call(
        _tc_head_body,
        grid=(grid,),
        in_specs=[
            pl.BlockSpec((_SC_BLOCK,), lambda i: (i,)),
            pl.BlockSpec((_NUM_TYPES, _SC_BLOCK), lambda i: (0, i)),
        ],
        out_specs=pl.BlockSpec((_NUM_TYPES, _SC_BLOCK), lambda i: (0, i)),
        out_shape=jax.ShapeDtypeStruct((_NUM_TYPES, _N_NODES), jnp.float32),
        input_output_aliases={1: 0},
    )(cls_head, out_tail)
    return out_t.T


# speedup vs baseline: 1.0720x; 1.0154x over previous
import functools

import jax
import jax.numpy as jnp
import numpy as np
from jax import lax
from jax.experimental import pallas as pl
from jax.experimental.pallas import tpu as pltpu
from jax.experimental.pallas import tpu_sc as plsc

_ATOMIC_NUMBERS = np.array(
    sorted({1, 2, 4, 5, 6, 7, 8, 9, 12, 14, 15, 16, 17, 18, 20, 22, 30, 31,
            32, 33, 34, 35}),
    dtype=np.int32,
)
_NUM_TYPES = 22
_SCALING = 1.5
_N_NODES = 100000

_SC_NODES = 4096
_SC_BLOCK = 4096
_NW = 32
_NODES_PW = _SC_NODES // _NW
_GROUPS_PW = _NODES_PW // 16

_LOOKUP = np.zeros((40,), dtype=np.int32)
_LOOKUP[_ATOMIC_NUMBERS] = np.arange(_NUM_TYPES, dtype=np.int32)


def _make_sc_lookup():
    mesh = plsc.VectorSubcoreMesh(core_axis_name="c", subcore_axis_name="s")

    @functools.partial(
        pl.kernel,
        mesh=mesh,
        out_type=jax.ShapeDtypeStruct((_SC_NODES,), jnp.int32),
        scratch_types=[
            pltpu.VMEM((_NODES_PW,), jnp.int32),
            pltpu.VMEM((40,), jnp.int32),
            pltpu.VMEM((_NODES_PW,), jnp.int32),
        ],
        compiler_params=pltpu.CompilerParams(needs_layout_passes=False),
    )
    def sc_lookup(node_hbm, tbl_hbm, cls_hbm, nt_v, tbl_v, cls_v):
        wid = lax.axis_index("s") * 2 + lax.axis_index("c")
        base = wid * _NODES_PW
        pltpu.sync_copy(node_hbm.at[pl.ds(base, _NODES_PW)], nt_v)
        pltpu.sync_copy(tbl_hbm, tbl_v)
        for j in range(_GROUPS_PW):
            o = j * 16
            z = nt_v[pl.ds(o, 16)] + 1
            cls_v[pl.ds(o, 16)] = plsc.load_gather(tbl_v, [z])
        pltpu.sync_copy(cls_v, cls_hbm.at[pl.ds(base, _NODES_PW)])

    return sc_lookup


_SC_LOOKUP = _make_sc_lookup()


def _tc_tail_body(nt_ref, atoms_ref, out_ref):
    z = nt_ref[...].reshape(1, _N_NODES) + 1
    atoms = atoms_ref[...]
    col = lax.broadcasted_iota(jnp.int32, (1, _N_NODES), 1)
    hit = (z == atoms) & (col >= _SC_NODES)
    out_ref[...] = jnp.where(hit, jnp.float32(_SCALING), jnp.float32(0.0))


def _tc_head_body(cls_ref, prev_ref, out_ref):
    del prev_ref
    cls = cls_ref[...].reshape(1, _SC_BLOCK)
    j = lax.broadcasted_iota(jnp.int32, (_NUM_TYPES, 1), 0)
    out_ref[...] = jnp.where(cls == j, jnp.float32(_SCALING), jnp.float32(0.0))


def kernel(node_type, pos):
    del pos
    nt = node_type.astype(jnp.int32).reshape(-1)
    tbl = jnp.asarray(_LOOKUP)
    atoms = jnp.asarray(_ATOMIC_NUMBERS).reshape(_NUM_TYPES, 1)

    cls_head = _SC_LOOKUP(nt, tbl)

    out_tail = pl.pallas_call(
        _tc_tail_body,
        out_shape=jax.ShapeDtypeStruct((_NUM_TYPES, _N_NODES), jnp.float32),
    )(nt, atoms)

    grid = _SC_NODES // _SC_BLOCK
    out_t = pl.pallas_call(
        _tc_head_body,
        grid=(grid,),
        in_specs=[
            pl.BlockSpec((_SC_BLOCK,), lambda i: (i,)),
            pl.BlockSpec((_NUM_TYPES, _SC_BLOCK), lambda i: (0, i)),
        ],
        out_specs=pl.BlockSpec((_NUM_TYPES, _SC_BLOCK), lambda i: (0, i)),
        out_shape=jax.ShapeDtypeStruct((_NUM_TYPES, _N_NODES), jnp.float32),
        input_output_aliases={1: 0},
    )(cls_head, out_tail)
    return out_t.T
